# baseline (speedup 1.0000x reference)
import functools

import jax
import jax.numpy as jnp
from jax import lax
from jax.experimental import pallas as pl
from jax.experimental.pallas import tpu as pltpu
from jax.experimental.pallas import tpu_sc as plsc

NC = 2
NS = 16
NW = NC * NS
LANES = 16

B, S, D = 4, 2048, 768
POS_PER_W = S // NW
K = 32
NCHUNK = B * (POS_PER_W // K)
COLS = D // LANES


def _embed_body(ids_hbm, wte_hbm, wpe_hbm, out_hbm,
                idx_v, buf_t, buf_p, sem_i, sem_p,
                sem_t0, sem_t1, sem_o0, sem_o1):
  cid = lax.axis_index("c")
  sid = lax.axis_index("s")
  wid = sid * NC + cid
  pos0 = wid * POS_PER_W

  sem_t = (sem_t0, sem_t1)
  sem_o = (sem_o0, sem_o1)

  cp_idx = pltpu.async_copy(ids_hbm.at[wid], idx_v, sem_i)
  cp_wpe = pltpu.async_copy(wpe_hbm.at[pl.ds(pos0, POS_PER_W)], buf_p, sem_p)
  cp_idx.wait()

  g_cur = pltpu.async_copy(wte_hbm.at[idx_v.at[0]], buf_t.at[0], sem_t[0])
  cp_wpe.wait()

  out_cp = [None, None]
  for c in range(NCHUNK):
    slot = c & 1
    h = c & 1
    b = c >> 1
    if c + 1 < NCHUNK:
      nslot = 1 - slot
      if out_cp[nslot] is not None:
        out_cp[nslot].wait()
      g_next = pltpu.async_copy(
          wte_hbm.at[idx_v.at[c + 1]], buf_t.at[nslot], sem_t[nslot])
    g_cur.wait()

    tref = buf_t.at[slot]

    def row_body(r, _, tref=tref, h=h):
      pr = r + h * K
      for j in range(COLS):
        sl = pl.ds(j * LANES, LANES)
        tref[r, sl] = tref[r, sl] + buf_p[pr, sl]
      return 0

    lax.fori_loop(0, K, row_body, 0)

    base = b * S + pos0 + h * K
    out_cp[slot] = pltpu.async_copy(
        buf_t.at[slot], out_hbm.at[pl.ds(base, K)], sem_o[slot])
    if c + 1 < NCHUNK:
      g_cur = g_next

  out_cp[0].wait()
  out_cp[1].wait()


@jax.jit
def _embed(ids_r, wte, wpe):
  mesh = plsc.VectorSubcoreMesh(core_axis_name="c", subcore_axis_name="s")
  f = pl.kernel(
      _embed_body,
      out_type=jax.ShapeDtypeStruct((B * S, D), jnp.float32),
      mesh=mesh,
      scratch_types=[
          pltpu.VMEM((NCHUNK, K), jnp.int32),
          pltpu.VMEM((2, K, D), jnp.float32),
          pltpu.VMEM((POS_PER_W, D), jnp.float32),
          pltpu.SemaphoreType.DMA,
          pltpu.SemaphoreType.DMA,
          pltpu.SemaphoreType.DMA,
          pltpu.SemaphoreType.DMA,
          pltpu.SemaphoreType.DMA,
          pltpu.SemaphoreType.DMA,
      ],
  )
  return f(ids_r, wte, wpe)


def kernel(input_ids, wte, wpe):
  ids = input_ids.astype(jnp.int32)
  ids_r = ids.reshape(B, NW, 2, K).transpose(1, 0, 2, 3).reshape(NW, NCHUNK, K)
  out = _embed(ids_r, wte, wpe)
  return out.reshape(B, S, D)


# speedup vs baseline: 1.0018x; 1.0018x over previous
import functools

import jax
import jax.numpy as jnp
from jax import lax
from jax.experimental import pallas as pl
from jax.experimental.pallas import tpu as pltpu
from jax.experimental.pallas import tpu_sc as plsc

NC = 2
NS = 16
NW = NC * NS
LANES = 16

B, S, D = 4, 2048, 768
POS_PER_W = S // NW
K = 32
NCHUNK = B * (POS_PER_W // K)
COLS = D // LANES
NBUF = 3


def _embed_body(ids_hbm, wte_hbm, wpe_hbm, out_hbm,
                idx_v, buf_t, buf_p, sem_i, sem_p,
                sem_t0, sem_t1, sem_t2, sem_o0, sem_o1, sem_o2):
  cid = lax.axis_index("c")
  sid = lax.axis_index("s")
  wid = sid * NC + cid
  pos0 = wid * POS_PER_W

  sem_t = (sem_t0, sem_t1, sem_t2)
  sem_o = (sem_o0, sem_o1, sem_o2)

  cp_idx = pltpu.async_copy(ids_hbm.at[wid], idx_v, sem_i)
  cp_wpe = pltpu.async_copy(wpe_hbm.at[pl.ds(pos0, POS_PER_W)], buf_p, sem_p)
  cp_idx.wait()

  def issue_gather(c):
    slot = c % NBUF
    return pltpu.async_copy(
        wte_hbm.at[idx_v.at[c]], buf_t.at[slot], sem_t[slot])

  g = {0: issue_gather(0), 1: issue_gather(1)}
  cp_wpe.wait()

  out_cp = {}
  for c in range(NCHUNK):
    slot = c % NBUF
    h = c & 1
    b = c >> 1
    if c + 2 < NCHUNK:
      if c - 1 >= 0:
        out_cp[c - 1].wait()
      g[c + 2] = issue_gather(c + 2)
    g[c].wait()

    tref = buf_t.at[slot]

    def row_body(r, _, tref=tref, h=h):
      pr = r + h * K
      for j in range(COLS):
        sl = pl.ds(j * LANES, LANES)
        tref[r, sl] = tref[r, sl] + buf_p[pr, sl]
      return 0

    lax.fori_loop(0, K, row_body, 0)

    base = b * S + pos0 + h * K
    out_cp[c] = pltpu.async_copy(
        buf_t.at[slot], out_hbm.at[pl.ds(base, K)], sem_o[slot])

  for c in range(NCHUNK - NBUF, NCHUNK):
    out_cp[c].wait()


@jax.jit
def _embed(ids_r, wte, wpe):
  mesh = plsc.VectorSubcoreMesh(core_axis_name="c", subcore_axis_name="s")
  f = pl.kernel(
      _embed_body,
      out_type=jax.ShapeDtypeStruct((B * S, D), jnp.float32),
      mesh=mesh,
      scratch_types=[
          pltpu.VMEM((NCHUNK, K), jnp.int32),
          pltpu.VMEM((NBUF, K, D), jnp.float32),
          pltpu.VMEM((POS_PER_W, D), jnp.float32),
          pltpu.SemaphoreType.DMA,
          pltpu.SemaphoreType.DMA,
          pltpu.SemaphoreType.DMA,
          pltpu.SemaphoreType.DMA,
          pltpu.SemaphoreType.DMA,
          pltpu.SemaphoreType.DMA,
          pltpu.SemaphoreType.DMA,
          pltpu.SemaphoreType.DMA,
      ],
  )
  return f(ids_r, wte, wpe)


def kernel(input_ids, wte, wpe):
  ids = input_ids.astype(jnp.int32)
  ids_r = ids.reshape(B, NW, 2, K).transpose(1, 0, 2, 3).reshape(NW, NCHUNK, K)
  out = _embed(ids_r, wte, wpe)
  return out.reshape(B, S, D)


# speedup vs baseline: 1.3728x; 1.3703x over previous
import functools

import jax
import jax.numpy as jnp
from jax import lax
from jax.experimental import pallas as pl
from jax.experimental.pallas import tpu as pltpu
from jax.experimental.pallas import tpu_sc as plsc

NC = 2
NS = 16
NW = NC * NS
LANES = 16

B, S, D = 4, 2048, 768
POS_PER_W = S // NW
K = 16
NQ = POS_PER_W // K
COLS = D // LANES
CHALF = COLS // 2


def _embed_body(ids_hbm, wte_hbm, wpe_hbm, out_hbm,
                idx_v, bufs, slabs, sem_i, sem_p,
                sem_g0, sem_g1, sem_o0, sem_o1):
  cid = lax.axis_index("c")
  sid = lax.axis_index("s")
  wid = sid * NC + cid
  pos0 = wid * POS_PER_W

  sem_g = (sem_g0, sem_g1)
  sem_o = (sem_o0, sem_o1)

  idx_cps = []
  for b in range(B):
    for q in range(NQ):
      idx_cps.append(pltpu.async_copy(
          ids_hbm.at[b, pl.ds(pos0 + q * K, K)], idx_v.at[b, q], sem_i))

  def issue_group(q):
    gp = q & 1
    slab_cp = pltpu.async_copy(
        wpe_hbm.at[pl.ds(pos0 + q * K, K)], slabs.at[gp], sem_p)
    g_cps = [
        pltpu.async_copy(
            wte_hbm.at[idx_v.at[b, q]], bufs.at[gp, b], sem_g[gp])
        for b in range(B)
    ]
    return (slab_cp, g_cps)

  for cp in idx_cps:
    cp.wait()

  pend = {0: issue_group(0), 1: issue_group(1)}
  wbs = {}
  for q in range(NQ):
    gp = q & 1
    slab_cp, g_cps = pend[q]
    slab_cp.wait()
    for cp in g_cps:
      cp.wait()

    def row_body(r, _, gp=gp):
      for half in range(2):
        base = half * CHALF * LANES
        wrow = [slabs[gp, r, pl.ds(base + j * LANES, LANES)]
                for j in range(CHALF)]
        for b in range(B):
          for j in range(CHALF):
            sl = pl.ds(base + j * LANES, LANES)
            bufs[gp, b, r, sl] = bufs[gp, b, r, sl] + wrow[j]
      return 0

    lax.fori_loop(0, K, row_body, 0)

    wbs[q] = [
        pltpu.async_copy(
            bufs.at[gp, b], out_hbm.at[pl.ds(b * S + pos0 + q * K, K)],
            sem_o[gp])
        for b in range(B)
    ]
    if q + 2 < NQ:
      for cp in wbs[q]:
        cp.wait()
      pend[q + 2] = issue_group(q + 2)

  for q in (NQ - 2, NQ - 1):
    for cp in wbs[q]:
      cp.wait()


@jax.jit
def _embed(ids, wte, wpe):
  mesh = plsc.VectorSubcoreMesh(core_axis_name="c", subcore_axis_name="s")
  f = pl.kernel(
      _embed_body,
      out_type=jax.ShapeDtypeStruct((B * S, D), jnp.float32),
      mesh=mesh,
      scratch_types=[
          pltpu.VMEM((B, NQ, K), jnp.int32),
          pltpu.VMEM((2, B, K, D), jnp.float32),
          pltpu.VMEM((2, K, D), jnp.float32),
          pltpu.SemaphoreType.DMA,
          pltpu.SemaphoreType.DMA,
          pltpu.SemaphoreType.DMA,
          pltpu.SemaphoreType.DMA,
          pltpu.SemaphoreType.DMA,
          pltpu.SemaphoreType.DMA,
      ],
  )
  return f(ids, wte, wpe)


def kernel(input_ids, wte, wpe):
  out = _embed(input_ids.astype(jnp.int32), wte, wpe)
  return out.reshape(B, S, D)
